# baseline (speedup 1.0000x reference)
import jax
import jax.numpy as jnp
from jax.experimental import pallas as pl
from jax.experimental.pallas import tpu as pltpu

N = 10000
HID = 128


def _bn(x, g, b, eps=1e-5):
    if x.ndim == 3:
        m = jnp.mean(x, axis=(0, 2), keepdims=True)
        v = jnp.var(x, axis=(0, 2), keepdims=True)
        return (x - m) / jnp.sqrt(v + eps) * g[None, :, None] + b[None, :, None]
    m = jnp.mean(x, axis=0, keepdims=True)
    v = jnp.var(x, axis=0, keepdims=True)
    return (x - m) / jnp.sqrt(v + eps) * g[None, :] + b[None, :]


def _matmul_kernel(x_ref, w_ref, o_ref):
    k = pl.program_id(1)

    @pl.when(k == 0)
    def _():
        o_ref[...] = jnp.zeros_like(o_ref)

    o_ref[...] += jnp.dot(x_ref[...], w_ref[...],
                          preferred_element_type=jnp.float32)


def _pallas_matmul(x, w, bm=512, bk=896):
    m, k = x.shape
    _, n = w.shape
    mp = ((m + bm - 1) // bm) * bm
    if mp != m:
        x = jnp.pad(x, ((0, mp - m), (0, 0)))
    grid = (mp // bm, k // bk)
    out = pl.pallas_call(
        _matmul_kernel,
        grid=grid,
        in_specs=[
            pl.BlockSpec((bm, bk), lambda i, j: (i, j)),
            pl.BlockSpec((bk, n), lambda i, j: (j, 0)),
        ],
        out_specs=pl.BlockSpec((bm, n), lambda i, j: (i, 0)),
        out_shape=jax.ShapeDtypeStruct((mp, n), jnp.float32),
    )(x, w)
    return out[:m]


def kernel(inputs, graph, conv0_w, conv0_b, bn_c0_g, bn_c0_b, ms0_w, ms0_b, bn_m0_g, bn_m0_b, sage0_ws, sage0_wn, sage0_bias, sage1_ws, sage1_wn, sage1_bias, sage2_ws, sage2_wn, sage2_bias, bn0_g, bn0_b, bn1_g, bn1_b, bn2_g, bn2_b, head_w1, head_b1, head_bng, head_bnb, head_w2, head_b2):
    h = inputs[0]
    y = jnp.einsum('ncl,oc->nol', h, conv0_w[:, :, 0]) + conv0_b[None, :, None]
    h = jax.nn.relu(_bn(y, bn_c0_g, bn_c0_b))
    cnn1 = h
    x = jnp.einsum('ncl,oc->nol', h, ms0_w[:, :, 0]) + ms0_b[None, :, None]
    x = jax.nn.relu(_bn(x, bn_m0_g, bn_m0_b))
    h = x + h
    cnn2 = h
    h = h.reshape(h.shape[0], -1)
    g = graph[0]
    src = g[:, 0]
    dst = g[:, 1]
    n = h.shape[0]

    deg = jax.ops.segment_sum(jnp.ones((src.shape[0],), jnp.float32), dst,
                              num_segments=n)
    inv_deg = 1.0 / jnp.clip(deg, 1.0)

    w_cat = jnp.concatenate([sage0_ws.T, sage0_wn.T], axis=1)
    proj = _pallas_matmul(h, w_cat)
    h_self = proj[:, :HID]
    hs = proj[:, HID:]
    agg = jax.ops.segment_sum(hs[src], dst, num_segments=n) * inv_deg[:, None]
    h = h_self + agg + sage0_bias[None, :]
    h = jax.nn.relu(_bn(h, bn0_g, bn0_b))

    for ws, wn, bi, bg, bb in ((sage1_ws, sage1_wn, sage1_bias, bn1_g, bn1_b),
                               (sage2_ws, sage2_wn, sage2_bias, bn2_g, bn2_b)):
        pre = h
        agg = jax.ops.segment_sum(h[src], dst, num_segments=n) * inv_deg[:, None]
        h = h @ ws.T + agg @ wn.T + bi[None, :]
        h = jax.nn.relu(_bn(h, bg, bb) + pre)
    fin = h
    h = h @ head_w1.T + head_b1[None, :]
    h = jax.nn.relu(_bn(h, head_bng, head_bnb))
    h = h @ head_w2.T + head_b2[None, :]
    return (h, (cnn1, cnn2, fin))


# speedup vs baseline: 2.1059x; 2.1059x over previous
import functools

import jax
import jax.numpy as jnp
from jax import lax
from jax.experimental import pallas as pl
from jax.experimental.pallas import tpu as pltpu
from jax.experimental.pallas import tpu_sc as plsc

N = 10000
E = 320000
HID = 128
L = 140
C0 = 5
CH = 64
NL = float(N * L)
EPS = 1e-5

TPC = 16
EPT = E // TPC
CHUNK = 128
NCHUNK = (EPT + CHUNK - 1) // CHUNK
EPT_PAD = NCHUNK * CHUNK
NPC_R = 5120
AGG_ROWS = NPC_R + 128
ZPS = AGG_ROWS // 16
OPS = NPC_R // 16


def _t1_body(x_ref, w0t_ref, o_ref):
    i = pl.program_id(0)
    x_b = x_ref[...]
    xt = jnp.swapaxes(x_b, 1, 2).reshape(-1, C0)
    h = jnp.dot(xt, w0t_ref[...], preferred_element_type=jnp.float32)
    s = jnp.sum(h, axis=0)
    q = jnp.sum(h * h, axis=0)
    st = jnp.stack([s, q], axis=0)

    @pl.when(i == 0)
    def _():
        o_ref[...] = jnp.zeros_like(o_ref)

    o_ref[...] += st


def _t2_body(x_ref, w0t_ref, st0_ref, g0_ref, b0_ref, w2_ref,
             cnn1_ref, st1_ref, bn):
    i = pl.program_id(0)
    st0 = st0_ref[...]
    m0 = st0[0] / NL
    v0 = st0[1] / NL - m0 * m0
    a0 = g0_ref[...] / jnp.sqrt(v0 + EPS)
    c0 = b0_ref[...] - m0 * a0
    x_b = x_ref[...]
    xt = jnp.swapaxes(x_b, 1, 2).reshape(-1, C0)
    h = jnp.dot(xt, w0t_ref[...], preferred_element_type=jnp.float32)
    ht = jax.nn.relu(h * a0[None, :] + c0[None, :])
    cnn1_ref[...] = jnp.swapaxes(ht.reshape(bn, L, CH), 1, 2)
    y2 = jnp.dot(ht, w2_ref[...], preferred_element_type=jnp.float32)
    s = jnp.sum(y2, axis=0)
    q = jnp.sum(y2 * y2, axis=0)
    st = jnp.stack([s, q], axis=0)

    @pl.when(i == 0)
    def _():
        st1_ref[...] = jnp.zeros_like(st1_ref)

    st1_ref[...] += st


def _t3_body(x_ref, w0t_ref, st0_ref, g0_ref, b0_ref, w2_ref,
             st1_ref, g1_ref, b1_ref, cnn2_ref, bn):
    st0 = st0_ref[...]
    m0 = st0[0] / NL
    v0 = st0[1] / NL - m0 * m0
    a0 = g0_ref[...] / jnp.sqrt(v0 + EPS)
    c0 = b0_ref[...] - m0 * a0
    st1 = st1_ref[...]
    m1 = st1[0] / NL
    v1 = st1[1] / NL - m1 * m1
    a1 = g1_ref[...] / jnp.sqrt(v1 + EPS)
    c1 = b1_ref[...] - m1 * a1
    x_b = x_ref[...]
    xt = jnp.swapaxes(x_b, 1, 2).reshape(-1, C0)
    h = jnp.dot(xt, w0t_ref[...], preferred_element_type=jnp.float32)
    ht = jax.nn.relu(h * a0[None, :] + c0[None, :])
    y2 = jnp.dot(ht, w2_ref[...], preferred_element_type=jnp.float32)
    c2 = jax.nn.relu(y2 * a1[None, :] + c1[None, :]) + ht
    cnn2_ref[...] = jnp.swapaxes(c2.reshape(bn, L, CH), 1, 2)


def _t4_body(x_ref, ws_ref, wn_ref, hself_ref, hs_ref):
    k = pl.program_id(1)

    @pl.when(k == 0)
    def _():
        hself_ref[...] = jnp.zeros_like(hself_ref)
        hs_ref[...] = jnp.zeros_like(hs_ref)

    x_b = x_ref[...]
    hself_ref[...] += jnp.dot(x_b, ws_ref[...],
                              preferred_element_type=jnp.float32)
    hs_ref[...] += jnp.dot(x_b, wn_ref[...],
                           preferred_element_type=jnp.float32)


def _zero_vmem(buf):
    zero16 = jnp.zeros((16,), jnp.float32)

    def zloop(i, _):
        buf[i // 8, pl.ds((i % 8) * 16, 16)] = zero16
        return 0

    lax.fori_loop(0, 1024, zloop, 0)


def _sc_agg_body(table, src_h, dst_h, agg_o,
                 src_v, dst_v, rows_v, zbuf, agg_sh, sem):
    cid = lax.axis_index("c")
    sid = lax.axis_index("s")
    _zero_vmem(zbuf)
    base = sid * ZPS
    pltpu.sync_copy(zbuf, agg_sh.at[pl.ds(base, 128)])
    pltpu.sync_copy(zbuf, agg_sh.at[pl.ds(base + 128, 128)])
    pltpu.sync_copy(zbuf.at[pl.ds(0, ZPS - 256)],
                    agg_sh.at[pl.ds(base + 256, ZPS - 256)])
    plsc.subcore_barrier()
    pltpu.sync_copy(src_h.at[sid], src_v)
    pltpu.sync_copy(dst_h.at[cid, sid], dst_v)

    def chunk(j, _):
        pltpu.async_copy(table.at[src_v.at[j]], rows_v, sem).wait()
        pltpu.sync_copy(rows_v, agg_sh.at[dst_v.at[j]], add=True)
        return 0

    lax.fori_loop(0, NCHUNK, chunk, 0)
    plsc.subcore_barrier()
    pltpu.sync_copy(agg_sh.at[pl.ds(sid * OPS, OPS)],
                    agg_o.at[cid, pl.ds(sid * OPS, OPS)])


_SC_MESH = dict(core_axis_name="c", subcore_axis_name="s")


def _sc_aggregate(table, src3, dst3c):
    k = pl.kernel(
        _sc_agg_body,
        out_type=[jax.ShapeDtypeStruct((2, NPC_R, HID), jnp.float32)],
        mesh=plsc.VectorSubcoreMesh(**_SC_MESH),
        scratch_types=[
            pltpu.VMEM((NCHUNK, CHUNK), jnp.int32),
            pltpu.VMEM((NCHUNK, CHUNK), jnp.int32),
            pltpu.VMEM((CHUNK, HID), jnp.float32),
            pltpu.VMEM((128, 128), jnp.float32),
            pltpu.VMEM_SHARED((AGG_ROWS, HID), jnp.float32),
            pltpu.SemaphoreType.DMA,
        ])
    return k(table, src3, dst3c)[0].reshape(2 * NPC_R, HID)


def _remap_body(dst_ref, o0_ref, o1_ref):
    d = dst_ref[...]
    o0_ref[...] = jnp.where(d < NPC_R, d, NPC_R)
    r = d - NPC_R
    o1_ref[...] = jnp.where((r >= 0) & (r < NPC_R), r, NPC_R)


def _e1_body(hself_ref, agg_ref, deg_ref, bias_ref, pre_ref, st_ref):
    i = pl.program_id(0)
    invd = 1.0 / jnp.maximum(deg_ref[...], 1.0)
    agg = agg_ref[...] * invd
    pre = hself_ref[...] + agg + bias_ref[...][None, :]
    pre_ref[...] = pre
    st = jnp.stack([jnp.sum(pre, axis=0), jnp.sum(pre * pre, axis=0)], 0)

    @pl.when(i == 0)
    def _():
        st_ref[...] = jnp.zeros_like(st_ref)

    st_ref[...] += st


def _bnrelu_body(pre_ref, st_ref, g_ref, b_ref, o_ref, nrows):
    st = st_ref[...]
    m = st[0] / nrows
    v = st[1] / nrows - m * m
    a = g_ref[...] / jnp.sqrt(v + EPS)
    c = b_ref[...] - m * a
    o_ref[...] = jax.nn.relu(pre_ref[...] * a[None, :] + c[None, :])


def _f1_body(h_ref, agg_ref, deg_ref, ws_ref, wn_ref, bias_ref,
             pre_ref, st_ref):
    i = pl.program_id(0)
    invd = 1.0 / jnp.maximum(deg_ref[...], 1.0)
    agg = agg_ref[...] * invd
    pre = (jnp.dot(h_ref[...], ws_ref[...], preferred_element_type=jnp.float32)
           + jnp.dot(agg, wn_ref[...], preferred_element_type=jnp.float32)
           + bias_ref[...][None, :])
    pre_ref[...] = pre
    st = jnp.stack([jnp.sum(pre, axis=0), jnp.sum(pre * pre, axis=0)], 0)

    @pl.when(i == 0)
    def _():
        st_ref[...] = jnp.zeros_like(st_ref)

    st_ref[...] += st


def _bnrelures_body(pre_ref, st_ref, g_ref, b_ref, res_ref, o_ref, nrows):
    st = st_ref[...]
    m = st[0] / nrows
    v = st[1] / nrows - m * m
    a = g_ref[...] / jnp.sqrt(v + EPS)
    c = b_ref[...] - m * a
    o_ref[...] = jax.nn.relu(pre_ref[...] * a[None, :] + c[None, :]
                             + res_ref[...])


def _h1_body(fin_ref, w1_ref, b1_ref, y_ref, st_ref):
    i = pl.program_id(0)
    y = jnp.dot(fin_ref[...], w1_ref[...],
                preferred_element_type=jnp.float32) + b1_ref[...][None, :]
    y_ref[...] = y
    st = jnp.stack([jnp.sum(y, axis=0), jnp.sum(y * y, axis=0)], 0)

    @pl.when(i == 0)
    def _():
        st_ref[...] = jnp.zeros_like(st_ref)

    st_ref[...] += st


def _h2_body(y_ref, st_ref, g_ref, b_ref, w2_ref, b2_ref, o_ref, nrows):
    st = st_ref[...]
    m = st[0] / nrows
    v = st[1] / nrows - m * m
    a = g_ref[...] / jnp.sqrt(v + EPS)
    c = b_ref[...] - m * a
    r = jax.nn.relu(y_ref[...] * a[None, :] + c[None, :])
    o_ref[...] = jnp.dot(r, w2_ref[...],
                         preferred_element_type=jnp.float32) + b2_ref[...][None, :]


def _rep(shape):
    return pl.BlockSpec(shape, lambda *_: tuple(0 for _ in shape))


def kernel(inputs, graph, conv0_w, conv0_b, bn_c0_g, bn_c0_b, ms0_w, ms0_b, bn_m0_g, bn_m0_b, sage0_ws, sage0_wn, sage0_bias, sage1_ws, sage1_wn, sage1_bias, sage2_ws, sage2_wn, sage2_bias, bn0_g, bn0_b, bn1_g, bn1_b, bn2_g, bn2_b, head_w1, head_b1, head_bng, head_bnb, head_w2, head_b2):
    x = inputs[0]
    w0t = conv0_w[:, :, 0].T
    w2m = ms0_w[:, :, 0].T

    bn_t = 40
    grid_t = N // bn_t

    st0 = pl.pallas_call(
        _t1_body,
        grid=(grid_t,),
        in_specs=[pl.BlockSpec((bn_t, C0, L), lambda i: (i, 0, 0)),
                  _rep((C0, CH))],
        out_specs=_rep((2, CH)),
        out_shape=jax.ShapeDtypeStruct((2, CH), jnp.float32),
    )(x, w0t)

    cnn1, st1 = pl.pallas_call(
        functools.partial(_t2_body, bn=bn_t),
        grid=(grid_t,),
        in_specs=[pl.BlockSpec((bn_t, C0, L), lambda i: (i, 0, 0)),
                  _rep((C0, CH)), _rep((2, CH)), _rep((CH,)), _rep((CH,)),
                  _rep((CH, CH))],
        out_specs=[pl.BlockSpec((bn_t, CH, L), lambda i: (i, 0, 0)),
                   _rep((2, CH))],
        out_shape=[jax.ShapeDtypeStruct((N, CH, L), jnp.float32),
                   jax.ShapeDtypeStruct((2, CH), jnp.float32)],
    )(x, w0t, st0, bn_c0_g, bn_c0_b, w2m)

    cnn2 = pl.pallas_call(
        functools.partial(_t3_body, bn=bn_t),
        grid=(grid_t,),
        in_specs=[pl.BlockSpec((bn_t, C0, L), lambda i: (i, 0, 0)),
                  _rep((C0, CH)), _rep((2, CH)), _rep((CH,)), _rep((CH,)),
                  _rep((CH, CH)), _rep((2, CH)), _rep((CH,)), _rep((CH,))],
        out_specs=pl.BlockSpec((bn_t, CH, L), lambda i: (i, 0, 0)),
        out_shape=jax.ShapeDtypeStruct((N, CH, L), jnp.float32),
    )(x, w0t, st0, bn_c0_g, bn_c0_b, w2m, st1, bn_m0_g, bn_m0_b)

    h2d = cnn2.reshape(N, CH * L)
    bm, bk = 2000, 1280
    hself, hs = pl.pallas_call(
        _t4_body,
        grid=(N // bm, (CH * L) // bk),
        in_specs=[pl.BlockSpec((bm, bk), lambda i, k: (i, k)),
                  pl.BlockSpec((bk, HID), lambda i, k: (k, 0)),
                  pl.BlockSpec((bk, HID), lambda i, k: (k, 0))],
        out_specs=[pl.BlockSpec((bm, HID), lambda i, k: (i, 0)),
                   pl.BlockSpec((bm, HID), lambda i, k: (i, 0))],
        out_shape=[jax.ShapeDtypeStruct((N, HID), jnp.float32),
                   jax.ShapeDtypeStruct((N, HID), jnp.float32)],
    )(h2d, sage0_ws.T, sage0_wn.T)

    g = graph[0]
    src3 = jnp.pad(g[:, 0].reshape(TPC, EPT),
                   ((0, 0), (0, EPT_PAD - EPT))).reshape(TPC, NCHUNK, CHUNK)
    dst2 = jnp.pad(g[:, 1].reshape(TPC, EPT),
                   ((0, 0), (0, EPT_PAD - EPT)),
                   constant_values=N).reshape(TPC * NCHUNK, CHUNK)

    d0, d1 = pl.pallas_call(
        _remap_body,
        out_shape=[jax.ShapeDtypeStruct((TPC * NCHUNK, CHUNK), jnp.int32)] * 2,
    )(dst2)
    dst3c = jnp.stack([d0.reshape(TPC, NCHUNK, CHUNK),
                       d1.reshape(TPC, NCHUNK, CHUNK)], axis=0)

    ones_table = jnp.ones((N, HID), jnp.float32)
    deg = _sc_aggregate(ones_table, src3, dst3c)[:N, :1]
    agg0 = _sc_aggregate(hs, src3, dst3c)[:N]

    bm2 = 1000
    grid2 = N // bm2
    blk = pl.BlockSpec((bm2, HID), lambda i: (i, 0))
    blkd = pl.BlockSpec((bm2, 1), lambda i: (i, 0))
    vec = _rep((HID,))
    stspec = _rep((2, HID))
    st_shape = jax.ShapeDtypeStruct((2, HID), jnp.float32)
    h_shape = jax.ShapeDtypeStruct((N, HID), jnp.float32)

    pre1, sst0 = pl.pallas_call(
        _e1_body,
        grid=(grid2,),
        in_specs=[blk, blk, blkd, vec],
        out_specs=[blk, stspec],
        out_shape=[h_shape, st_shape],
    )(hself, agg0, deg, sage0_bias)

    h1 = pl.pallas_call(
        functools.partial(_bnrelu_body, nrows=float(N)),
        grid=(grid2,),
        in_specs=[blk, stspec, vec, vec],
        out_specs=blk,
        out_shape=h_shape,
    )(pre1, sst0, bn0_g, bn0_b)

    hcur = h1
    for ws, wn, bi, bg, bb in ((sage1_ws, sage1_wn, sage1_bias, bn1_g, bn1_b),
                               (sage2_ws, sage2_wn, sage2_bias, bn2_g, bn2_b)):
        agg_l = _sc_aggregate(hcur, src3, dst3c)[:N]
        pre, sst = pl.pallas_call(
            _f1_body,
            grid=(grid2,),
            in_specs=[blk, blk, blkd, _rep((HID, HID)), _rep((HID, HID)),
                      vec],
            out_specs=[blk, stspec],
            out_shape=[h_shape, st_shape],
        )(hcur, agg_l, deg, ws.T, wn.T, bi)
        hcur = pl.pallas_call(
            functools.partial(_bnrelures_body, nrows=float(N)),
            grid=(grid2,),
            in_specs=[blk, stspec, vec, vec, blk],
            out_specs=blk,
            out_shape=h_shape,
        )(pre, sst, bg, bb, hcur)

    fin = hcur

    vec64 = _rep((CH,))
    y, hst = pl.pallas_call(
        _h1_body,
        grid=(grid2,),
        in_specs=[blk, _rep((HID, CH)), vec64],
        out_specs=[pl.BlockSpec((bm2, CH), lambda i: (i, 0)), _rep((2, CH))],
        out_shape=[jax.ShapeDtypeStruct((N, CH), jnp.float32),
                   jax.ShapeDtypeStruct((2, CH), jnp.float32)],
    )(fin, head_w1.T, head_b1)

    out = pl.pallas_call(
        functools.partial(_h2_body, nrows=float(N)),
        grid=(grid2,),
        in_specs=[pl.BlockSpec((bm2, CH), lambda i: (i, 0)), _rep((2, CH)),
                  vec64, vec64, _rep((CH, 2)), _rep((2,))],
        out_specs=pl.BlockSpec((bm2, 2), lambda i: (i, 0)),
        out_shape=jax.ShapeDtypeStruct((N, 2), jnp.float32),
    )(y, hst, head_bng, head_bnb, head_w2.T, head_b2)

    return (out, (cnn1, cnn2, fin))


# speedup vs baseline: 2.3468x; 1.1144x over previous
import functools

import jax
import jax.numpy as jnp
from jax import lax
from jax.experimental import pallas as pl
from jax.experimental.pallas import tpu as pltpu
from jax.experimental.pallas import tpu_sc as plsc

N = 10000
E = 320000
HID = 128
L = 140
C0 = 5
CH = 64
NL = float(N * L)
EPS = 1e-5

TPC = 16
EPT = E // TPC
CHUNK = 128
NCHUNK = (EPT + CHUNK - 1) // CHUNK
EPT_PAD = NCHUNK * CHUNK
NPC_R = 5120
AGG_ROWS = NPC_R + 128
ZPS = AGG_ROWS // 16
OPS = NPC_R // 16


def _t1_body(x_ref, w0t_ref, o_ref):
    i = pl.program_id(0)
    x_b = x_ref[...]
    xt = jnp.swapaxes(x_b, 1, 2).reshape(-1, C0)
    h = jnp.dot(xt, w0t_ref[...], preferred_element_type=jnp.float32)
    s = jnp.sum(h, axis=0)
    q = jnp.sum(h * h, axis=0)
    st = jnp.stack([s, q], axis=0)

    @pl.when(i == 0)
    def _():
        o_ref[...] = jnp.zeros_like(o_ref)

    o_ref[...] += st


def _t2_body(x_ref, w0t_ref, st0_ref, g0_ref, b0_ref, w2_ref,
             cnn1_ref, st1_ref, bn):
    i = pl.program_id(0)
    st0 = st0_ref[...]
    m0 = st0[0] / NL
    v0 = st0[1] / NL - m0 * m0
    a0 = g0_ref[...] / jnp.sqrt(v0 + EPS)
    c0 = b0_ref[...] - m0 * a0
    x_b = x_ref[...]
    xt = jnp.swapaxes(x_b, 1, 2).reshape(-1, C0)
    h = jnp.dot(xt, w0t_ref[...], preferred_element_type=jnp.float32)
    ht = jax.nn.relu(h * a0[None, :] + c0[None, :])
    cnn1_ref[...] = jnp.swapaxes(ht.reshape(bn, L, CH), 1, 2)
    y2 = jnp.dot(ht, w2_ref[...], preferred_element_type=jnp.float32)
    s = jnp.sum(y2, axis=0)
    q = jnp.sum(y2 * y2, axis=0)
    st = jnp.stack([s, q], axis=0)

    @pl.when(i == 0)
    def _():
        st1_ref[...] = jnp.zeros_like(st1_ref)

    st1_ref[...] += st


def _t3_body(x_ref, w0t_ref, st0_ref, g0_ref, b0_ref, w2_ref,
             st1_ref, g1_ref, b1_ref, cnn2_ref, bn):
    st0 = st0_ref[...]
    m0 = st0[0] / NL
    v0 = st0[1] / NL - m0 * m0
    a0 = g0_ref[...] / jnp.sqrt(v0 + EPS)
    c0 = b0_ref[...] - m0 * a0
    st1 = st1_ref[...]
    m1 = st1[0] / NL
    v1 = st1[1] / NL - m1 * m1
    a1 = g1_ref[...] / jnp.sqrt(v1 + EPS)
    c1 = b1_ref[...] - m1 * a1
    x_b = x_ref[...]
    xt = jnp.swapaxes(x_b, 1, 2).reshape(-1, C0)
    h = jnp.dot(xt, w0t_ref[...], preferred_element_type=jnp.float32)
    ht = jax.nn.relu(h * a0[None, :] + c0[None, :])
    y2 = jnp.dot(ht, w2_ref[...], preferred_element_type=jnp.float32)
    c2 = jax.nn.relu(y2 * a1[None, :] + c1[None, :]) + ht
    cnn2_ref[...] = jnp.swapaxes(c2.reshape(bn, L, CH), 1, 2)


def _t4_body(x_ref, ws_ref, wn_ref, hself_ref, hs_ref):
    k = pl.program_id(1)

    @pl.when(k == 0)
    def _():
        hself_ref[...] = jnp.zeros_like(hself_ref)
        hs_ref[...] = jnp.zeros_like(hs_ref)

    accs = hself_ref[...]
    accn = hs_ref[...]
    for c in range(8):
        x_c = x_ref[:, c, :]
        accs += jnp.dot(x_c, ws_ref[c], preferred_element_type=jnp.float32)
        accn += jnp.dot(x_c, wn_ref[c], preferred_element_type=jnp.float32)
    hself_ref[...] = accs
    hs_ref[...] = accn


def _zero_vmem(buf):
    zero16 = jnp.zeros((16,), jnp.float32)

    def zloop(i, _):
        buf[i // 8, pl.ds((i % 8) * 16, 16)] = zero16
        return 0

    lax.fori_loop(0, 1024, zloop, 0)


def _zero_my_slice(sh, base, zbuf):
    pltpu.sync_copy(zbuf, sh.at[pl.ds(base, 128)])
    pltpu.sync_copy(zbuf, sh.at[pl.ds(base + 128, 128)])
    pltpu.sync_copy(zbuf.at[pl.ds(0, ZPS - 256)],
                    sh.at[pl.ds(base + 256, ZPS - 256)])


def _sc_agg_body(table, src_h, dst_h, agg_o, src_v, dst_v,
                 rows_a, rows_b, zbuf, agg_sh, sem_a, sem_b):
    cid = lax.axis_index("c")
    sid = lax.axis_index("s")
    _zero_vmem(zbuf)
    _zero_my_slice(agg_sh, sid * ZPS, zbuf)
    plsc.subcore_barrier()
    pltpu.sync_copy(src_h.at[sid], src_v)
    pltpu.sync_copy(dst_h.at[cid, sid], dst_v)

    def chunk(j, _):
        pltpu.async_copy(table.at[src_v.at[j]], rows_a, sem_a).wait()
        pltpu.sync_copy(rows_a, agg_sh.at[dst_v.at[j]], add=True)
        return 0

    lax.fori_loop(0, NCHUNK, chunk, 0)
    plsc.subcore_barrier()
    pltpu.sync_copy(agg_sh.at[pl.ds(sid * OPS, OPS)],
                    agg_o.at[cid, pl.ds(sid * OPS, OPS)])


def _sc_deg_body(dst_h, deg_o, dst_v, ones_v, zbuf, deg_sh, sem):
    cid = lax.axis_index("c")
    sid = lax.axis_index("s")
    _zero_vmem(zbuf)
    one16 = jnp.ones((16,), jnp.float32)

    def oloop(i, _):
        ones_v[i // 8, pl.ds((i % 8) * 16, 16)] = one16
        return 0

    lax.fori_loop(0, 1024, oloop, 0)
    _zero_my_slice(deg_sh, sid * ZPS, zbuf)
    plsc.subcore_barrier()
    pltpu.sync_copy(dst_h.at[cid, sid], dst_v)

    def chunk(j, _):
        pltpu.sync_copy(ones_v, deg_sh.at[dst_v.at[j]], add=True)
        return 0

    lax.fori_loop(0, NCHUNK, chunk, 0)
    plsc.subcore_barrier()
    pltpu.sync_copy(deg_sh.at[pl.ds(sid * OPS, OPS)],
                    deg_o.at[cid, pl.ds(sid * OPS, OPS)])


_SC_MESH = dict(core_axis_name="c", subcore_axis_name="s")


def _sc_aggregate(table, src3, dst3c):
    k = pl.kernel(
        _sc_agg_body,
        out_type=[jax.ShapeDtypeStruct((2, NPC_R, HID), jnp.float32)],
        mesh=plsc.VectorSubcoreMesh(**_SC_MESH),
        scratch_types=[
            pltpu.VMEM((NCHUNK, CHUNK), jnp.int32),
            pltpu.VMEM((NCHUNK, CHUNK), jnp.int32),
            pltpu.VMEM((CHUNK, HID), jnp.float32),
            pltpu.VMEM((CHUNK, HID), jnp.float32),
            pltpu.VMEM((128, 128), jnp.float32),
            pltpu.VMEM_SHARED((AGG_ROWS, HID), jnp.float32),
            pltpu.SemaphoreType.DMA,
            pltpu.SemaphoreType.DMA,
        ])
    return k(table, src3, dst3c)[0].reshape(2 * NPC_R, HID)


def _sc_degree(dst3c):
    k = pl.kernel(
        _sc_deg_body,
        out_type=[jax.ShapeDtypeStruct((2, NPC_R, HID), jnp.float32)],
        mesh=plsc.VectorSubcoreMesh(**_SC_MESH),
        scratch_types=[
            pltpu.VMEM((NCHUNK, CHUNK), jnp.int32),
            pltpu.VMEM((CHUNK, HID), jnp.float32),
            pltpu.VMEM((128, 128), jnp.float32),
            pltpu.VMEM_SHARED((AGG_ROWS, HID), jnp.float32),
            pltpu.SemaphoreType.DMA,
        ])
    return k(dst3c)[0].reshape(2 * NPC_R, HID)


def _remap_body(dst_ref, o_ref):
    d = dst_ref[...]
    o_ref[0] = jnp.where(d < NPC_R, d, NPC_R)
    r = d - NPC_R
    o_ref[1] = jnp.where((r >= 0) & (r < NPC_R), r, NPC_R)


def _e1_body(hself_ref, agg_ref, deg_ref, bias_ref, pre_ref, st_ref):
    i = pl.program_id(0)
    invd = 1.0 / jnp.maximum(deg_ref[...][:, :1], 1.0)
    agg = agg_ref[...] * invd
    pre = hself_ref[...] + agg + bias_ref[...][None, :]
    pre_ref[...] = pre
    st = jnp.stack([jnp.sum(pre, axis=0), jnp.sum(pre * pre, axis=0)], 0)

    @pl.when(i == 0)
    def _():
        st_ref[...] = jnp.zeros_like(st_ref)

    st_ref[...] += st


def _bnrelu_body(pre_ref, st_ref, g_ref, b_ref, o_ref, nrows):
    st = st_ref[...]
    m = st[0] / nrows
    v = st[1] / nrows - m * m
    a = g_ref[...] / jnp.sqrt(v + EPS)
    c = b_ref[...] - m * a
    o_ref[...] = jax.nn.relu(pre_ref[...] * a[None, :] + c[None, :])


def _f1_body(h_ref, agg_ref, deg_ref, ws_ref, wn_ref, bias_ref,
             pre_ref, st_ref):
    i = pl.program_id(0)
    invd = 1.0 / jnp.maximum(deg_ref[...][:, :1], 1.0)
    agg = agg_ref[...] * invd
    pre = (jnp.dot(h_ref[...], ws_ref[...], preferred_element_type=jnp.float32)
           + jnp.dot(agg, wn_ref[...], preferred_element_type=jnp.float32)
           + bias_ref[...][None, :])
    pre_ref[...] = pre
    st = jnp.stack([jnp.sum(pre, axis=0), jnp.sum(pre * pre, axis=0)], 0)

    @pl.when(i == 0)
    def _():
        st_ref[...] = jnp.zeros_like(st_ref)

    st_ref[...] += st


def _bnrelures_body(pre_ref, st_ref, g_ref, b_ref, res_ref, o_ref, nrows):
    st = st_ref[...]
    m = st[0] / nrows
    v = st[1] / nrows - m * m
    a = g_ref[...] / jnp.sqrt(v + EPS)
    c = b_ref[...] - m * a
    o_ref[...] = jax.nn.relu(pre_ref[...] * a[None, :] + c[None, :]
                             + res_ref[...])


def _h1_body(fin_ref, w1_ref, b1_ref, y_ref, st_ref):
    i = pl.program_id(0)
    y = jnp.dot(fin_ref[...], w1_ref[...],
                preferred_element_type=jnp.float32) + b1_ref[...][None, :]
    y_ref[...] = y
    st = jnp.stack([jnp.sum(y, axis=0), jnp.sum(y * y, axis=0)], 0)

    @pl.when(i == 0)
    def _():
        st_ref[...] = jnp.zeros_like(st_ref)

    st_ref[...] += st


def _h2_body(y_ref, st_ref, g_ref, b_ref, w2_ref, b2_ref, o_ref, nrows):
    st = st_ref[...]
    m = st[0] / nrows
    v = st[1] / nrows - m * m
    a = g_ref[...] / jnp.sqrt(v + EPS)
    c = b_ref[...] - m * a
    r = jax.nn.relu(y_ref[...] * a[None, :] + c[None, :])
    o_ref[...] = jnp.dot(r, w2_ref[...],
                         preferred_element_type=jnp.float32) + b2_ref[...][None, :]


def _rep(shape):
    return pl.BlockSpec(shape, lambda *_: tuple(0 for _ in shape))


def kernel(inputs, graph, conv0_w, conv0_b, bn_c0_g, bn_c0_b, ms0_w, ms0_b, bn_m0_g, bn_m0_b, sage0_ws, sage0_wn, sage0_bias, sage1_ws, sage1_wn, sage1_bias, sage2_ws, sage2_wn, sage2_bias, bn0_g, bn0_b, bn1_g, bn1_b, bn2_g, bn2_b, head_w1, head_b1, head_bng, head_bnb, head_w2, head_b2):
    x = inputs[0]
    w0t = conv0_w[:, :, 0].T
    w2m = ms0_w[:, :, 0].T

    bn_t = 40
    grid_t = N // bn_t

    st0 = pl.pallas_call(
        _t1_body,
        grid=(grid_t,),
        in_specs=[pl.BlockSpec((bn_t, C0, L), lambda i: (i, 0, 0)),
                  _rep((C0, CH))],
        out_specs=_rep((2, CH)),
        out_shape=jax.ShapeDtypeStruct((2, CH), jnp.float32),
    )(x, w0t)

    cnn1, st1 = pl.pallas_call(
        functools.partial(_t2_body, bn=bn_t),
        grid=(grid_t,),
        in_specs=[pl.BlockSpec((bn_t, C0, L), lambda i: (i, 0, 0)),
                  _rep((C0, CH)), _rep((2, CH)), _rep((CH,)), _rep((CH,)),
                  _rep((CH, CH))],
        out_specs=[pl.BlockSpec((bn_t, CH, L), lambda i: (i, 0, 0)),
                   _rep((2, CH))],
        out_shape=[jax.ShapeDtypeStruct((N, CH, L), jnp.float32),
                   jax.ShapeDtypeStruct((2, CH), jnp.float32)],
    )(x, w0t, st0, bn_c0_g, bn_c0_b, w2m)

    cnn2 = pl.pallas_call(
        functools.partial(_t3_body, bn=bn_t),
        grid=(grid_t,),
        in_specs=[pl.BlockSpec((bn_t, C0, L), lambda i: (i, 0, 0)),
                  _rep((C0, CH)), _rep((2, CH)), _rep((CH,)), _rep((CH,)),
                  _rep((CH, CH)), _rep((2, CH)), _rep((CH,)), _rep((CH,))],
        out_specs=pl.BlockSpec((bn_t, CH, L), lambda i: (i, 0, 0)),
        out_shape=jax.ShapeDtypeStruct((N, CH, L), jnp.float32),
    )(x, w0t, st0, bn_c0_g, bn_c0_b, w2m, st1, bn_m0_g, bn_m0_b)

    ws3 = sage0_ws.T.reshape(CH, L, HID)
    wn3 = sage0_wn.T.reshape(CH, L, HID)
    bm = 2000
    wspec = pl.BlockSpec((8, L, HID), lambda i, k: (k, 0, 0))
    hself, hs = pl.pallas_call(
        _t4_body,
        grid=(N // bm, CH // 8),
        in_specs=[pl.BlockSpec((bm, 8, L), lambda i, k: (i, k, 0)),
                  wspec, wspec],
        out_specs=[pl.BlockSpec((bm, HID), lambda i, k: (i, 0))] * 2,
        out_shape=[jax.ShapeDtypeStruct((N, HID), jnp.float32)] * 2,
    )(cnn2, ws3, wn3)

    g = graph[0]
    src3 = jnp.pad(g[:, 0].reshape(TPC, EPT),
                   ((0, 0), (0, EPT_PAD - EPT))).reshape(TPC, NCHUNK, CHUNK)
    dst2 = jnp.pad(g[:, 1].reshape(TPC, EPT),
                   ((0, 0), (0, EPT_PAD - EPT)),
                   constant_values=N).reshape(TPC * NCHUNK, CHUNK)

    dst3c = pl.pallas_call(
        _remap_body,
        out_shape=jax.ShapeDtypeStruct((2, TPC * NCHUNK, CHUNK), jnp.int32),
    )(dst2).reshape(2, TPC, NCHUNK, CHUNK)

    ones_table = jnp.ones((N, HID), jnp.float32)
    deg = _sc_aggregate(ones_table, src3, dst3c)[:N]
    agg0 = _sc_aggregate(hs, src3, dst3c)[:N]

    bm2 = 1000
    grid2 = N // bm2
    blk = pl.BlockSpec((bm2, HID), lambda i: (i, 0))
    blkd = blk
    vec = _rep((HID,))
    stspec = _rep((2, HID))
    st_shape = jax.ShapeDtypeStruct((2, HID), jnp.float32)
    h_shape = jax.ShapeDtypeStruct((N, HID), jnp.float32)

    pre1, sst0 = pl.pallas_call(
        _e1_body,
        grid=(grid2,),
        in_specs=[blk, blk, blkd, vec],
        out_specs=[blk, stspec],
        out_shape=[h_shape, st_shape],
    )(hself, agg0, deg, sage0_bias)

    h1 = pl.pallas_call(
        functools.partial(_bnrelu_body, nrows=float(N)),
        grid=(grid2,),
        in_specs=[blk, stspec, vec, vec],
        out_specs=blk,
        out_shape=h_shape,
    )(pre1, sst0, bn0_g, bn0_b)

    hcur = h1
    for ws, wn, bi, bg, bb in ((sage1_ws, sage1_wn, sage1_bias, bn1_g, bn1_b),
                               (sage2_ws, sage2_wn, sage2_bias, bn2_g, bn2_b)):
        agg_l = _sc_aggregate(hcur, src3, dst3c)[:N]
        pre, sst = pl.pallas_call(
            _f1_body,
            grid=(grid2,),
            in_specs=[blk, blk, blkd, _rep((HID, HID)), _rep((HID, HID)),
                      vec],
            out_specs=[blk, stspec],
            out_shape=[h_shape, st_shape],
        )(hcur, agg_l, deg, ws.T, wn.T, bi)
        hcur = pl.pallas_call(
            functools.partial(_bnrelures_body, nrows=float(N)),
            grid=(grid2,),
            in_specs=[blk, stspec, vec, vec, blk],
            out_specs=blk,
            out_shape=h_shape,
        )(pre, sst, bg, bb, hcur)

    fin = hcur

    vec64 = _rep((CH,))
    y, hst = pl.pallas_call(
        _h1_body,
        grid=(grid2,),
        in_specs=[blk, _rep((HID, CH)), vec64],
        out_specs=[pl.BlockSpec((bm2, CH), lambda i: (i, 0)), _rep((2, CH))],
        out_shape=[jax.ShapeDtypeStruct((N, CH), jnp.float32),
                   jax.ShapeDtypeStruct((2, CH), jnp.float32)],
    )(fin, head_w1.T, head_b1)

    out = pl.pallas_call(
        functools.partial(_h2_body, nrows=float(N)),
        grid=(grid2,),
        in_specs=[pl.BlockSpec((bm2, CH), lambda i: (i, 0)), _rep((2, CH)),
                  vec64, vec64, _rep((CH, 2)), _rep((2,))],
        out_specs=pl.BlockSpec((bm2, 2), lambda i: (i, 0)),
        out_shape=jax.ShapeDtypeStruct((N, 2), jnp.float32),
    )(y, hst, head_bng, head_bnb, head_w2.T, head_b2)

    return (out, (cnn1, cnn2, fin))


# speedup vs baseline: 2.3723x; 1.0109x over previous
import functools

import jax
import jax.numpy as jnp
from jax import lax
from jax.experimental import pallas as pl
from jax.experimental.pallas import tpu as pltpu
from jax.experimental.pallas import tpu_sc as plsc

N = 10000
E = 320000
HID = 128
L = 140
C0 = 5
CH = 64
NL = float(N * L)
EPS = 1e-5

TPC = 16
EPT = E // TPC
CHUNK = 128
NCHUNK = (EPT + CHUNK - 1) // CHUNK
EPT_PAD = NCHUNK * CHUNK
NPC_R = 5120
AGG_ROWS = NPC_R + 128
ZPS = AGG_ROWS // 16
OPS = NPC_R // 16


def _t1_body(x_ref, w0t_ref, o_ref):
    i = pl.program_id(0)
    x_b = x_ref[...]
    xt = jnp.swapaxes(x_b, 1, 2).reshape(-1, C0)
    h = jnp.dot(xt, w0t_ref[...], preferred_element_type=jnp.float32)
    s = jnp.sum(h, axis=0)
    q = jnp.sum(h * h, axis=0)
    st = jnp.stack([s, q], axis=0)

    @pl.when(i == 0)
    def _():
        o_ref[...] = jnp.zeros_like(o_ref)

    o_ref[...] += st


def _t2_body(x_ref, w0t_ref, st0_ref, g0_ref, b0_ref, w2_ref,
             cnn1_ref, st1_ref, bn):
    i = pl.program_id(0)
    st0 = st0_ref[...]
    m0 = st0[0] / NL
    v0 = st0[1] / NL - m0 * m0
    a0 = g0_ref[...] / jnp.sqrt(v0 + EPS)
    c0 = b0_ref[...] - m0 * a0
    x_b = x_ref[...]
    xt = jnp.swapaxes(x_b, 1, 2).reshape(-1, C0)
    h = jnp.dot(xt, w0t_ref[...], preferred_element_type=jnp.float32)
    ht = jax.nn.relu(h * a0[None, :] + c0[None, :])
    cnn1_ref[...] = jnp.swapaxes(ht.reshape(bn, L, CH), 1, 2)
    y2 = jnp.dot(ht, w2_ref[...], preferred_element_type=jnp.float32)
    s = jnp.sum(y2, axis=0)
    q = jnp.sum(y2 * y2, axis=0)
    st = jnp.stack([s, q], axis=0)

    @pl.when(i == 0)
    def _():
        st1_ref[...] = jnp.zeros_like(st1_ref)

    st1_ref[...] += st


def _t3_body(x_ref, w0t_ref, st0_ref, g0_ref, b0_ref, w2_ref,
             st1_ref, g1_ref, b1_ref, cnn2_ref, bn):
    st0 = st0_ref[...]
    m0 = st0[0] / NL
    v0 = st0[1] / NL - m0 * m0
    a0 = g0_ref[...] / jnp.sqrt(v0 + EPS)
    c0 = b0_ref[...] - m0 * a0
    st1 = st1_ref[...]
    m1 = st1[0] / NL
    v1 = st1[1] / NL - m1 * m1
    a1 = g1_ref[...] / jnp.sqrt(v1 + EPS)
    c1 = b1_ref[...] - m1 * a1
    x_b = x_ref[...]
    xt = jnp.swapaxes(x_b, 1, 2).reshape(-1, C0)
    h = jnp.dot(xt, w0t_ref[...], preferred_element_type=jnp.float32)
    ht = jax.nn.relu(h * a0[None, :] + c0[None, :])
    y2 = jnp.dot(ht, w2_ref[...], preferred_element_type=jnp.float32)
    c2 = jax.nn.relu(y2 * a1[None, :] + c1[None, :]) + ht
    cnn2_ref[...] = jnp.swapaxes(c2.reshape(bn, L, CH), 1, 2)


def _t4_body(x_ref, ws_ref, wn_ref, hself_ref, hs_ref):
    k = pl.program_id(1)

    @pl.when(k == 0)
    def _():
        hself_ref[...] = jnp.zeros_like(hself_ref)
        hs_ref[...] = jnp.zeros_like(hs_ref)

    accs = hself_ref[...]
    accn = hs_ref[...]
    for c in range(8):
        x_c = x_ref[:, c, :]
        accs += jnp.dot(x_c, ws_ref[c], preferred_element_type=jnp.float32)
        accn += jnp.dot(x_c, wn_ref[c], preferred_element_type=jnp.float32)
    hself_ref[...] = accs
    hs_ref[...] = accn


def _zero_vmem(buf):
    zero16 = jnp.zeros((16,), jnp.float32)

    def zloop(i, _):
        buf[i // 8, pl.ds((i % 8) * 16, 16)] = zero16
        return 0

    lax.fori_loop(0, 1024, zloop, 0)


def _zero_my_slice(sh, base, zbuf):
    pltpu.sync_copy(zbuf, sh.at[pl.ds(base, 128)])
    pltpu.sync_copy(zbuf, sh.at[pl.ds(base + 128, 128)])
    pltpu.sync_copy(zbuf.at[pl.ds(0, ZPS - 256)],
                    sh.at[pl.ds(base + 256, ZPS - 256)])


def _sc_agg_body(table, src_h, dst_h, agg_o, src_v, dst_v,
                 rows_a, rows_b, zbuf, agg_sh, sem_a, sem_b):
    cid = lax.axis_index("c")
    sid = lax.axis_index("s")
    _zero_vmem(zbuf)
    _zero_my_slice(agg_sh, sid * ZPS, zbuf)
    plsc.subcore_barrier()
    pltpu.sync_copy(src_h.at[sid], src_v)
    pltpu.sync_copy(dst_h.at[cid, sid], dst_v)

    def chunk(j, _):
        pltpu.async_copy(table.at[src_v.at[j]], rows_a, sem_a).wait()
        pltpu.sync_copy(rows_a, agg_sh.at[dst_v.at[j]], add=True)
        return 0

    lax.fori_loop(0, NCHUNK, chunk, 0)
    plsc.subcore_barrier()
    pltpu.sync_copy(agg_sh.at[pl.ds(sid * OPS, OPS)],
                    agg_o.at[cid, pl.ds(sid * OPS, OPS)])


def _sc_deg_body(dst_h, deg_o, dst_v, ones_v, zbuf, deg_sh, sem):
    cid = lax.axis_index("c")
    sid = lax.axis_index("s")
    _zero_vmem(zbuf)
    one16 = jnp.ones((16,), jnp.float32)

    def oloop(i, _):
        ones_v[i // 8, pl.ds((i % 8) * 16, 16)] = one16
        return 0

    lax.fori_loop(0, 1024, oloop, 0)
    _zero_my_slice(deg_sh, sid * ZPS, zbuf)
    plsc.subcore_barrier()
    pltpu.sync_copy(dst_h.at[cid, sid], dst_v)

    def chunk(j, _):
        pltpu.sync_copy(ones_v, deg_sh.at[dst_v.at[j]], add=True)
        return 0

    lax.fori_loop(0, NCHUNK, chunk, 0)
    plsc.subcore_barrier()
    pltpu.sync_copy(deg_sh.at[pl.ds(sid * OPS, OPS)],
                    deg_o.at[cid, pl.ds(sid * OPS, OPS)])


_SC_MESH = dict(core_axis_name="c", subcore_axis_name="s")


def _sc_aggregate(table, src3, dst3c):
    k = pl.kernel(
        _sc_agg_body,
        out_type=[jax.ShapeDtypeStruct((2, NPC_R, HID), jnp.float32)],
        mesh=plsc.VectorSubcoreMesh(**_SC_MESH),
        scratch_types=[
            pltpu.VMEM((NCHUNK, CHUNK), jnp.int32),
            pltpu.VMEM((NCHUNK, CHUNK), jnp.int32),
            pltpu.VMEM((CHUNK, HID), jnp.float32),
            pltpu.VMEM((CHUNK, HID), jnp.float32),
            pltpu.VMEM((128, 128), jnp.float32),
            pltpu.VMEM_SHARED((AGG_ROWS, HID), jnp.float32),
            pltpu.SemaphoreType.DMA,
            pltpu.SemaphoreType.DMA,
        ])
    return k(table, src3, dst3c)[0].reshape(2 * NPC_R, HID)


def _sc_degree(dst3c):
    k = pl.kernel(
        _sc_deg_body,
        out_type=[jax.ShapeDtypeStruct((2, NPC_R, HID), jnp.float32)],
        mesh=plsc.VectorSubcoreMesh(**_SC_MESH),
        scratch_types=[
            pltpu.VMEM((NCHUNK, CHUNK), jnp.int32),
            pltpu.VMEM((CHUNK, HID), jnp.float32),
            pltpu.VMEM((128, 128), jnp.float32),
            pltpu.VMEM_SHARED((AGG_ROWS, HID), jnp.float32),
            pltpu.SemaphoreType.DMA,
        ])
    return k(dst3c)[0].reshape(2 * NPC_R, HID)


def _remap_body(dst_ref, o_ref):
    d = dst_ref[...]
    o_ref[0] = jnp.where(d < NPC_R, d, NPC_R)
    r = d - NPC_R
    o_ref[1] = jnp.where((r >= 0) & (r < NPC_R), r, NPC_R)


def _e1_body(hself_ref, agg_ref, deg_ref, bias_ref, pre_ref, st_ref):
    i = pl.program_id(0)
    invd = 1.0 / jnp.maximum(deg_ref[...][:, :1], 1.0)
    agg = agg_ref[...] * invd
    pre = hself_ref[...] + agg + bias_ref[...][None, :]
    pre_ref[...] = pre
    st = jnp.stack([jnp.sum(pre, axis=0), jnp.sum(pre * pre, axis=0)], 0)

    @pl.when(i == 0)
    def _():
        st_ref[...] = jnp.zeros_like(st_ref)

    st_ref[...] += st


def _bnrelu_body(pre_ref, st_ref, g_ref, b_ref, o_ref, nrows):
    st = st_ref[...]
    m = st[0] / nrows
    v = st[1] / nrows - m * m
    a = g_ref[...] / jnp.sqrt(v + EPS)
    c = b_ref[...] - m * a
    o_ref[...] = jax.nn.relu(pre_ref[...] * a[None, :] + c[None, :])


def _f1_body(h_ref, agg_ref, deg_ref, ws_ref, wn_ref, bias_ref,
             pre_ref, st_ref):
    i = pl.program_id(0)
    invd = 1.0 / jnp.maximum(deg_ref[...][:, :1], 1.0)
    agg = agg_ref[...] * invd
    pre = (jnp.dot(h_ref[...], ws_ref[...], preferred_element_type=jnp.float32)
           + jnp.dot(agg, wn_ref[...], preferred_element_type=jnp.float32)
           + bias_ref[...][None, :])
    pre_ref[...] = pre
    st = jnp.stack([jnp.sum(pre, axis=0), jnp.sum(pre * pre, axis=0)], 0)

    @pl.when(i == 0)
    def _():
        st_ref[...] = jnp.zeros_like(st_ref)

    st_ref[...] += st


def _bnrelures_body(pre_ref, st_ref, g_ref, b_ref, res_ref, o_ref, nrows):
    st = st_ref[...]
    m = st[0] / nrows
    v = st[1] / nrows - m * m
    a = g_ref[...] / jnp.sqrt(v + EPS)
    c = b_ref[...] - m * a
    o_ref[...] = jax.nn.relu(pre_ref[...] * a[None, :] + c[None, :]
                             + res_ref[...])


def _h1_body(fin_ref, w1_ref, b1_ref, y_ref, st_ref):
    i = pl.program_id(0)
    y = jnp.dot(fin_ref[...], w1_ref[...],
                preferred_element_type=jnp.float32) + b1_ref[...][None, :]
    y_ref[...] = y
    st = jnp.stack([jnp.sum(y, axis=0), jnp.sum(y * y, axis=0)], 0)

    @pl.when(i == 0)
    def _():
        st_ref[...] = jnp.zeros_like(st_ref)

    st_ref[...] += st


def _h2_body(y_ref, st_ref, g_ref, b_ref, w2_ref, b2_ref, o_ref, nrows):
    st = st_ref[...]
    m = st[0] / nrows
    v = st[1] / nrows - m * m
    a = g_ref[...] / jnp.sqrt(v + EPS)
    c = b_ref[...] - m * a
    r = jax.nn.relu(y_ref[...] * a[None, :] + c[None, :])
    o_ref[...] = jnp.dot(r, w2_ref[...],
                         preferred_element_type=jnp.float32) + b2_ref[...][None, :]


def _rep(shape):
    return pl.BlockSpec(shape, lambda *_: tuple(0 for _ in shape))


def kernel(inputs, graph, conv0_w, conv0_b, bn_c0_g, bn_c0_b, ms0_w, ms0_b, bn_m0_g, bn_m0_b, sage0_ws, sage0_wn, sage0_bias, sage1_ws, sage1_wn, sage1_bias, sage2_ws, sage2_wn, sage2_bias, bn0_g, bn0_b, bn1_g, bn1_b, bn2_g, bn2_b, head_w1, head_b1, head_bng, head_bnb, head_w2, head_b2):
    x = inputs[0]
    w0t = conv0_w[:, :, 0].T
    w2m = ms0_w[:, :, 0].T

    bn_t = 80
    grid_t = N // bn_t

    st0 = pl.pallas_call(
        _t1_body,
        grid=(grid_t,),
        in_specs=[pl.BlockSpec((bn_t, C0, L), lambda i: (i, 0, 0)),
                  _rep((C0, CH))],
        out_specs=_rep((2, CH)),
        out_shape=jax.ShapeDtypeStruct((2, CH), jnp.float32),
    )(x, w0t)

    cnn1, st1 = pl.pallas_call(
        functools.partial(_t2_body, bn=bn_t),
        grid=(grid_t,),
        in_specs=[pl.BlockSpec((bn_t, C0, L), lambda i: (i, 0, 0)),
                  _rep((C0, CH)), _rep((2, CH)), _rep((CH,)), _rep((CH,)),
                  _rep((CH, CH))],
        out_specs=[pl.BlockSpec((bn_t, CH, L), lambda i: (i, 0, 0)),
                   _rep((2, CH))],
        out_shape=[jax.ShapeDtypeStruct((N, CH, L), jnp.float32),
                   jax.ShapeDtypeStruct((2, CH), jnp.float32)],
    )(x, w0t, st0, bn_c0_g, bn_c0_b, w2m)

    cnn2 = pl.pallas_call(
        functools.partial(_t3_body, bn=bn_t),
        grid=(grid_t,),
        in_specs=[pl.BlockSpec((bn_t, C0, L), lambda i: (i, 0, 0)),
                  _rep((C0, CH)), _rep((2, CH)), _rep((CH,)), _rep((CH,)),
                  _rep((CH, CH)), _rep((2, CH)), _rep((CH,)), _rep((CH,))],
        out_specs=pl.BlockSpec((bn_t, CH, L), lambda i: (i, 0, 0)),
        out_shape=jax.ShapeDtypeStruct((N, CH, L), jnp.float32),
    )(x, w0t, st0, bn_c0_g, bn_c0_b, w2m, st1, bn_m0_g, bn_m0_b)

    ws3 = sage0_ws.T.reshape(CH, L, HID)
    wn3 = sage0_wn.T.reshape(CH, L, HID)
    bm = 2000
    wspec = pl.BlockSpec((8, L, HID), lambda i, k: (k, 0, 0))
    hself, hs = pl.pallas_call(
        _t4_body,
        grid=(N // bm, CH // 8),
        in_specs=[pl.BlockSpec((bm, 8, L), lambda i, k: (i, k, 0)),
                  wspec, wspec],
        out_specs=[pl.BlockSpec((bm, HID), lambda i, k: (i, 0))] * 2,
        out_shape=[jax.ShapeDtypeStruct((N, HID), jnp.float32)] * 2,
    )(cnn2, ws3, wn3)

    g = graph[0]
    src3 = jnp.pad(g[:, 0].reshape(TPC, EPT),
                   ((0, 0), (0, EPT_PAD - EPT))).reshape(TPC, NCHUNK, CHUNK)
    dst2 = jnp.pad(g[:, 1].reshape(TPC, EPT),
                   ((0, 0), (0, EPT_PAD - EPT)),
                   constant_values=N).reshape(TPC * NCHUNK, CHUNK)

    dst3c = pl.pallas_call(
        _remap_body,
        out_shape=jax.ShapeDtypeStruct((2, TPC * NCHUNK, CHUNK), jnp.int32),
    )(dst2).reshape(2, TPC, NCHUNK, CHUNK)

    ones_table = jnp.ones((N, HID), jnp.float32)
    deg = _sc_aggregate(ones_table, src3, dst3c)[:N]
    agg0 = _sc_aggregate(hs, src3, dst3c)[:N]

    bm2 = 1000
    grid2 = N // bm2
    blk = pl.BlockSpec((bm2, HID), lambda i: (i, 0))
    blkd = blk
    vec = _rep((HID,))
    stspec = _rep((2, HID))
    st_shape = jax.ShapeDtypeStruct((2, HID), jnp.float32)
    h_shape = jax.ShapeDtypeStruct((N, HID), jnp.float32)

    pre1, sst0 = pl.pallas_call(
        _e1_body,
        grid=(grid2,),
        in_specs=[blk, blk, blkd, vec],
        out_specs=[blk, stspec],
        out_shape=[h_shape, st_shape],
    )(hself, agg0, deg, sage0_bias)

    h1 = pl.pallas_call(
        functools.partial(_bnrelu_body, nrows=float(N)),
        grid=(grid2,),
        in_specs=[blk, stspec, vec, vec],
        out_specs=blk,
        out_shape=h_shape,
    )(pre1, sst0, bn0_g, bn0_b)

    hcur = h1
    for ws, wn, bi, bg, bb in ((sage1_ws, sage1_wn, sage1_bias, bn1_g, bn1_b),
                               (sage2_ws, sage2_wn, sage2_bias, bn2_g, bn2_b)):
        agg_l = _sc_aggregate(hcur, src3, dst3c)[:N]
        pre, sst = pl.pallas_call(
            _f1_body,
            grid=(grid2,),
            in_specs=[blk, blk, blkd, _rep((HID, HID)), _rep((HID, HID)),
                      vec],
            out_specs=[blk, stspec],
            out_shape=[h_shape, st_shape],
        )(hcur, agg_l, deg, ws.T, wn.T, bi)
        hcur = pl.pallas_call(
            functools.partial(_bnrelures_body, nrows=float(N)),
            grid=(grid2,),
            in_specs=[blk, stspec, vec, vec, blk],
            out_specs=blk,
            out_shape=h_shape,
        )(pre, sst, bg, bb, hcur)

    fin = hcur

    vec64 = _rep((CH,))
    y, hst = pl.pallas_call(
        _h1_body,
        grid=(grid2,),
        in_specs=[blk, _rep((HID, CH)), vec64],
        out_specs=[pl.BlockSpec((bm2, CH), lambda i: (i, 0)), _rep((2, CH))],
        out_shape=[jax.ShapeDtypeStruct((N, CH), jnp.float32),
                   jax.ShapeDtypeStruct((2, CH), jnp.float32)],
    )(fin, head_w1.T, head_b1)

    out = pl.pallas_call(
        functools.partial(_h2_body, nrows=float(N)),
        grid=(grid2,),
        in_specs=[pl.BlockSpec((bm2, CH), lambda i: (i, 0)), _rep((2, CH)),
                  vec64, vec64, _rep((CH, 2)), _rep((2,))],
        out_specs=pl.BlockSpec((bm2, 2), lambda i: (i, 0)),
        out_shape=jax.ShapeDtypeStruct((N, 2), jnp.float32),
    )(y, hst, head_bng, head_bnb, head_w2.T, head_b2)

    return (out, (cnn1, cnn2, fin))
